# initial kernel scaffold (unmeasured)
import math

import jax
import jax.numpy as jnp
from jax import lax
from jax.experimental import pallas as pl
from jax.experimental.pallas import tpu as pltpu

N_DEV = 4
SQ = 2048
D = 1024
HQ = 8
DH = 128
QB = 512
SCALE = 0.08838834764831843


def _rope_tables(off):
    ri = lax.broadcasted_iota(jnp.float32, (SQ, DH), 0)
    ci = lax.broadcasted_iota(jnp.int32, (SQ, DH), 1)
    f = (ci // 2).astype(jnp.float32)
    inv = jnp.exp(f * (-math.log(10000.0) / (DH // 2)))
    ang = (off.astype(jnp.float32) + ri) * inv
    return jnp.cos(ang), jnp.sin(ang)


def _rot_mat():
    i = lax.broadcasted_iota(jnp.int32, (DH, DH), 0)
    j = lax.broadcasted_iota(jnp.int32, (DH, DH), 1)
    plus = (j == i + 1) & (i % 2 == 0)
    minus = (j == i - 1) & (i % 2 == 1)
    return plus.astype(jnp.float32) - minus.astype(jnp.float32)


def _qkv_body(x_ref, wq_ref, wk_ref, wv_ref, q_ref, k_ref, v_ref):
    off = lax.axis_index("i") * SQ
    cos, sin = _rope_tables(off)
    rot = _rot_mat()
    x = x_ref[...]

    def proj_rope(w_ref):
        t = jnp.dot(x, w_ref[...], preferred_element_type=jnp.float32)
        tr = jnp.dot(t, rot, preferred_element_type=jnp.float32)
        return t * cos + tr * sin

    q_ref[...] = proj_rope(wq_ref) * SCALE
    k_ref[...] = proj_rope(wk_ref)
    v_ref[...] = jnp.dot(x, wv_ref[...], preferred_element_type=jnp.float32)


def _qkv(x, Wq, Wk, Wv):
    return pl.pallas_call(
        _qkv_body,
        grid=(HQ,),
        in_specs=[
            pl.BlockSpec((SQ, D), lambda h: (0, 0)),
            pl.BlockSpec((D, DH), lambda h: (0, h)),
            pl.BlockSpec((D, DH), lambda h: (0, h)),
            pl.BlockSpec((D, DH), lambda h: (0, h)),
        ],
        out_specs=[pl.BlockSpec((SQ, DH), lambda h: (0, h))] * 3,
        out_shape=[jax.ShapeDtypeStruct((SQ, D), jnp.float32)] * 3,
    )(x, Wq, Wk, Wv)


def _ring_body(k_ref, v_ref, kf_ref, vf_ref, copy_sems, sk, rk, sv, rv):
    my = lax.axis_index("i")
    left = lax.rem(my + N_DEV - 1, N_DEV)
    right = lax.rem(my + 1, N_DEV)

    barrier = pltpu.get_barrier_semaphore()
    for nbr in (left, right):
        pl.semaphore_signal(
            barrier, inc=1, device_id=(nbr,),
            device_id_type=pl.DeviceIdType.MESH,
        )
    pl.semaphore_wait(barrier, 2)

    ck = pltpu.make_async_copy(
        k_ref, kf_ref.at[pl.ds(my * SQ, SQ), :], copy_sems.at[0])
    cv = pltpu.make_async_copy(
        v_ref, vf_ref.at[pl.ds(my * SQ, SQ), :], copy_sems.at[1])
    ck.start()
    cv.start()
    ck.wait()
    cv.wait()

    for h in range(N_DEV - 1):
        src = lax.rem(my + N_DEV - h, N_DEV)
        sl = pl.ds(src * SQ, SQ)
        rdma_k = pltpu.make_async_remote_copy(
            src_ref=kf_ref.at[sl, :],
            dst_ref=kf_ref.at[sl, :],
            send_sem=sk.at[h],
            recv_sem=rk.at[h],
            device_id=(right,),
            device_id_type=pl.DeviceIdType.MESH,
        )
        rdma_v = pltpu.make_async_remote_copy(
            src_ref=vf_ref.at[sl, :],
            dst_ref=vf_ref.at[sl, :],
            send_sem=sv.at[h],
            recv_sem=rv.at[h],
            device_id=(right,),
            device_id_type=pl.DeviceIdType.MESH,
        )
        rdma_k.start()
        rdma_v.start()
        rdma_k.wait()
        rdma_v.wait()


def _ring_gather(k, v):
    return pl.pallas_call(
        _ring_body,
        in_specs=[pl.BlockSpec(memory_space=pltpu.VMEM)] * 2,
        out_specs=[pl.BlockSpec(memory_space=pltpu.ANY)] * 2,
        out_shape=[jax.ShapeDtypeStruct((N_DEV * SQ, D), jnp.float32)] * 2,
        scratch_shapes=[
            pltpu.SemaphoreType.DMA((2,)),
            pltpu.SemaphoreType.DMA((3,)),
            pltpu.SemaphoreType.DMA((3,)),
            pltpu.SemaphoreType.DMA((3,)),
            pltpu.SemaphoreType.DMA((3,)),
        ],
        compiler_params=pltpu.CompilerParams(collective_id=0),
    )(k, v)


def _attn_body(q_ref, k_ref, v_ref, o_ref):
    s = lax.dot_general(
        q_ref[...], k_ref[...],
        (((1,), (1,)), ((), ())),
        preferred_element_type=jnp.float32,
    )
    m = jnp.max(s, axis=1, keepdims=True)
    p = jnp.exp(s - m)
    l = jnp.sum(p, axis=1, keepdims=True)
    o = jnp.dot(p, v_ref[...], preferred_element_type=jnp.float32)
    o_ref[...] = o / l


def _attn(q, kf, vf):
    return pl.pallas_call(
        _attn_body,
        grid=(HQ, SQ // QB),
        in_specs=[
            pl.BlockSpec((QB, DH), lambda h, qb: (qb, h)),
            pl.BlockSpec((N_DEV * SQ, DH), lambda h, qb: (0, h)),
            pl.BlockSpec((N_DEV * SQ, DH), lambda h, qb: (0, h)),
        ],
        out_specs=pl.BlockSpec((QB, DH), lambda h, qb: (qb, h)),
        out_shape=jax.ShapeDtypeStruct((SQ, D), jnp.float32),
    )(q, kf, vf)


def _proj_body(c_ref, w_ref, o_ref):
    o_ref[...] = jnp.dot(
        c_ref[...], w_ref[...], preferred_element_type=jnp.float32)


def _proj(ctx, Wo):
    return pl.pallas_call(
        _proj_body,
        out_shape=jax.ShapeDtypeStruct((SQ, D), jnp.float32),
    )(ctx, Wo)


def kernel(x, Wq, Wk, Wv, Wo):
    x2 = x.reshape(SQ, D)
    q, k, v = _qkv(x2, Wq, Wk, Wv)
    kf, vf = _ring_gather(k, v)
    ctx = _attn(q, kf, vf)
    out = _proj(ctx, Wo)
    return out.reshape(1, SQ, D)


# baseline (device time: 1033624 ns/iter reference)
import math

import jax
import jax.numpy as jnp
from jax import lax
from jax.experimental import pallas as pl
from jax.experimental.pallas import tpu as pltpu

N_DEV = 4
SQ = 2048
D = 1024
HQ = 8
DH = 128
QB = 256
SCALE = 0.08838834764831843


def _rope_tables(off):
    ri = lax.broadcasted_iota(jnp.int32, (SQ, DH), 0).astype(jnp.float32)
    ci = lax.broadcasted_iota(jnp.int32, (SQ, DH), 1)
    f = (ci // 2).astype(jnp.float32)
    inv = jnp.exp(f * (-math.log(10000.0) / (DH // 2)))
    ang = (off.astype(jnp.float32) + ri) * inv
    return jnp.cos(ang), jnp.sin(ang)


def _rot_mat():
    i = lax.broadcasted_iota(jnp.int32, (DH, DH), 0)
    j = lax.broadcasted_iota(jnp.int32, (DH, DH), 1)
    plus = (j == i + 1) & (i % 2 == 0)
    minus = (j == i - 1) & (i % 2 == 1)
    return plus.astype(jnp.float32) - minus.astype(jnp.float32)


def _qkv_body(x_ref, wq_ref, wk_ref, wv_ref, q_ref, k_ref, v_ref):
    off = lax.axis_index("i") * SQ
    cos, sin = _rope_tables(off)
    rot = _rot_mat()
    x = x_ref[...]

    def proj_rope(w_ref):
        t = jnp.dot(x, w_ref[...], preferred_element_type=jnp.float32)
        tr = jnp.dot(t, rot, preferred_element_type=jnp.float32)
        return t * cos + tr * sin

    q_ref[...] = proj_rope(wq_ref) * SCALE
    k_ref[...] = proj_rope(wk_ref)
    v_ref[...] = jnp.dot(x, wv_ref[...], preferred_element_type=jnp.float32)


def _qkv(x, Wq, Wk, Wv):
    return pl.pallas_call(
        _qkv_body,
        grid=(HQ,),
        in_specs=[
            pl.BlockSpec((SQ, D), lambda h: (0, 0)),
            pl.BlockSpec((D, DH), lambda h: (0, h)),
            pl.BlockSpec((D, DH), lambda h: (0, h)),
            pl.BlockSpec((D, DH), lambda h: (0, h)),
        ],
        out_specs=[pl.BlockSpec((SQ, DH), lambda h: (0, h))] * 3,
        out_shape=[jax.ShapeDtypeStruct((SQ, D), jnp.float32)] * 3,
    )(x, Wq, Wk, Wv)


def _ring_body(k_ref, v_ref, kf_ref, vf_ref, copy_sems, sk, rk, sv, rv):
    my = lax.axis_index("i")
    left = lax.rem(my + N_DEV - 1, N_DEV)
    right = lax.rem(my + 1, N_DEV)

    barrier = pltpu.get_barrier_semaphore()
    for nbr in (left, right):
        pl.semaphore_signal(
            barrier, inc=1, device_id=(nbr,),
            device_id_type=pl.DeviceIdType.MESH,
        )
    pl.semaphore_wait(barrier, 2)

    ck = pltpu.make_async_copy(
        k_ref, kf_ref.at[pl.ds(my * SQ, SQ), :], copy_sems.at[0])
    cv = pltpu.make_async_copy(
        v_ref, vf_ref.at[pl.ds(my * SQ, SQ), :], copy_sems.at[1])
    ck.start()
    cv.start()
    ck.wait()
    cv.wait()

    for h in range(N_DEV - 1):
        src = lax.rem(my + N_DEV - h, N_DEV)
        sl = pl.ds(src * SQ, SQ)
        rdma_k = pltpu.make_async_remote_copy(
            src_ref=kf_ref.at[sl, :],
            dst_ref=kf_ref.at[sl, :],
            send_sem=sk.at[h],
            recv_sem=rk.at[h],
            device_id=(right,),
            device_id_type=pl.DeviceIdType.MESH,
        )
        rdma_v = pltpu.make_async_remote_copy(
            src_ref=vf_ref.at[sl, :],
            dst_ref=vf_ref.at[sl, :],
            send_sem=sv.at[h],
            recv_sem=rv.at[h],
            device_id=(right,),
            device_id_type=pl.DeviceIdType.MESH,
        )
        rdma_k.start()
        rdma_v.start()
        rdma_k.wait()
        rdma_v.wait()


def _ring_gather(k, v):
    return pl.pallas_call(
        _ring_body,
        in_specs=[pl.BlockSpec(memory_space=pltpu.VMEM)] * 2,
        out_specs=[pl.BlockSpec(memory_space=pl.ANY)] * 2,
        out_shape=[jax.ShapeDtypeStruct((N_DEV * SQ, D), jnp.float32)] * 2,
        scratch_shapes=[
            pltpu.SemaphoreType.DMA((2,)),
            pltpu.SemaphoreType.DMA((3,)),
            pltpu.SemaphoreType.DMA((3,)),
            pltpu.SemaphoreType.DMA((3,)),
            pltpu.SemaphoreType.DMA((3,)),
        ],
        compiler_params=pltpu.CompilerParams(collective_id=0),
    )(k, v)


def _attn_body(q_ref, k_ref, v_ref, o_ref):
    s = lax.dot_general(
        q_ref[...], k_ref[...],
        (((1,), (1,)), ((), ())),
        preferred_element_type=jnp.float32,
    )
    m = jnp.max(s, axis=1, keepdims=True)
    p = jnp.exp(s - m)
    l = jnp.sum(p, axis=1, keepdims=True)
    o = jnp.dot(p, v_ref[...], preferred_element_type=jnp.float32)
    o_ref[...] = o / l


def _attn(q, kf, vf):
    return pl.pallas_call(
        _attn_body,
        grid=(HQ, SQ // QB),
        in_specs=[
            pl.BlockSpec((QB, DH), lambda h, qb: (qb, h)),
            pl.BlockSpec((N_DEV * SQ, DH), lambda h, qb: (0, h)),
            pl.BlockSpec((N_DEV * SQ, DH), lambda h, qb: (0, h)),
        ],
        out_specs=pl.BlockSpec((QB, DH), lambda h, qb: (qb, h)),
        out_shape=jax.ShapeDtypeStruct((SQ, D), jnp.float32),
        compiler_params=pltpu.CompilerParams(
            vmem_limit_bytes=60 * 1024 * 1024),
    )(q, kf, vf)


def _proj_body(c_ref, w_ref, o_ref):
    o_ref[...] = jnp.dot(
        c_ref[...], w_ref[...], preferred_element_type=jnp.float32)


def _proj(ctx, Wo):
    return pl.pallas_call(
        _proj_body,
        out_shape=jax.ShapeDtypeStruct((SQ, D), jnp.float32),
    )(ctx, Wo)


def kernel(x, Wq, Wk, Wv, Wo):
    x2 = x.reshape(SQ, D)
    q, k, v = _qkv(x2, Wq, Wk, Wv)
    kf, vf = _ring_gather(k, v)
    ctx = _attn(q, kf, vf)
    out = _proj(ctx, Wo)
    return out.reshape(1, SQ, D)


# device time: 629776 ns/iter; 1.6413x vs baseline; 1.6413x over previous
import math

import jax
import jax.numpy as jnp
from jax import lax
from jax.experimental import pallas as pl
from jax.experimental.pallas import tpu as pltpu

N_DEV = 4
SQ = 2048
D = 1024
HQ = 8
DH = 128
QB = 256
SCALE = 0.08838834764831843


def _rope_tables(off):
    ri = lax.broadcasted_iota(jnp.int32, (SQ, DH), 0).astype(jnp.float32)
    ci = lax.broadcasted_iota(jnp.int32, (SQ, DH), 1)
    f = (ci // 2).astype(jnp.float32)
    inv = jnp.exp(f * (-math.log(10000.0) / (DH // 2)))
    ang = (off.astype(jnp.float32) + ri) * inv
    return jnp.cos(ang), jnp.sin(ang)


def _rot_mat():
    i = lax.broadcasted_iota(jnp.int32, (DH, DH), 0)
    j = lax.broadcasted_iota(jnp.int32, (DH, DH), 1)
    plus = (j == i + 1) & (i % 2 == 0)
    minus = (j == i - 1) & (i % 2 == 1)
    return plus.astype(jnp.float32) - minus.astype(jnp.float32)


def _qkv_body(x_ref, wq_ref, wk_ref, wv_ref, q_ref, k_ref, v_ref):
    off = lax.axis_index("i") * SQ
    cos, sin = _rope_tables(off)
    rot = _rot_mat().astype(jnp.bfloat16)
    x = x_ref[...].astype(jnp.bfloat16)

    def proj_rope(w_ref):
        t = jnp.dot(x, w_ref[...].astype(jnp.bfloat16),
                    preferred_element_type=jnp.float32)
        tr = jnp.dot(t.astype(jnp.bfloat16), rot,
                     preferred_element_type=jnp.float32)
        return t * cos + tr * sin

    q_ref[...] = (proj_rope(wq_ref) * SCALE).astype(jnp.bfloat16)
    k_ref[...] = proj_rope(wk_ref).astype(jnp.bfloat16)
    v_ref[...] = jnp.dot(x, wv_ref[...].astype(jnp.bfloat16),
                         preferred_element_type=jnp.float32
                         ).astype(jnp.bfloat16)


def _qkv(x, Wq, Wk, Wv):
    return pl.pallas_call(
        _qkv_body,
        grid=(HQ,),
        in_specs=[
            pl.BlockSpec((SQ, D), lambda h: (0, 0)),
            pl.BlockSpec((D, DH), lambda h: (0, h)),
            pl.BlockSpec((D, DH), lambda h: (0, h)),
            pl.BlockSpec((D, DH), lambda h: (0, h)),
        ],
        out_specs=[pl.BlockSpec((SQ, DH), lambda h: (0, h))] * 3,
        out_shape=[jax.ShapeDtypeStruct((SQ, D), jnp.bfloat16)] * 3,
    )(x, Wq, Wk, Wv)


def _ring_body(k_ref, v_ref, kf_ref, vf_ref, copy_sems, sk, rk, sv, rv):
    my = lax.axis_index("i")
    left = lax.rem(my + N_DEV - 1, N_DEV)
    right = lax.rem(my + 1, N_DEV)

    barrier = pltpu.get_barrier_semaphore()
    for nbr in (left, right):
        pl.semaphore_signal(
            barrier, inc=1, device_id=(nbr,),
            device_id_type=pl.DeviceIdType.MESH,
        )
    pl.semaphore_wait(barrier, 2)

    ck = pltpu.make_async_copy(
        k_ref, kf_ref.at[pl.ds(my * SQ, SQ), :], copy_sems.at[0])
    cv = pltpu.make_async_copy(
        v_ref, vf_ref.at[pl.ds(my * SQ, SQ), :], copy_sems.at[1])
    ck.start()
    cv.start()
    ck.wait()
    cv.wait()

    for h in range(N_DEV - 1):
        src = lax.rem(my + N_DEV - h, N_DEV)
        sl = pl.ds(src * SQ, SQ)
        rdma_k = pltpu.make_async_remote_copy(
            src_ref=kf_ref.at[sl, :],
            dst_ref=kf_ref.at[sl, :],
            send_sem=sk.at[h],
            recv_sem=rk.at[h],
            device_id=(right,),
            device_id_type=pl.DeviceIdType.MESH,
        )
        rdma_v = pltpu.make_async_remote_copy(
            src_ref=vf_ref.at[sl, :],
            dst_ref=vf_ref.at[sl, :],
            send_sem=sv.at[h],
            recv_sem=rv.at[h],
            device_id=(right,),
            device_id_type=pl.DeviceIdType.MESH,
        )
        rdma_k.start()
        rdma_v.start()
        rdma_k.wait()
        rdma_v.wait()


def _ring_gather(k, v):
    return pl.pallas_call(
        _ring_body,
        in_specs=[pl.BlockSpec(memory_space=pltpu.VMEM)] * 2,
        out_specs=[pl.BlockSpec(memory_space=pl.ANY)] * 2,
        out_shape=[jax.ShapeDtypeStruct((N_DEV * SQ, D), jnp.bfloat16)] * 2,
        scratch_shapes=[
            pltpu.SemaphoreType.DMA((2,)),
            pltpu.SemaphoreType.DMA((3,)),
            pltpu.SemaphoreType.DMA((3,)),
            pltpu.SemaphoreType.DMA((3,)),
            pltpu.SemaphoreType.DMA((3,)),
        ],
        compiler_params=pltpu.CompilerParams(collective_id=0),
    )(k, v)


def _attn_body(q_ref, k_ref, v_ref, o_ref):
    s = lax.dot_general(
        q_ref[...], k_ref[...],
        (((1,), (1,)), ((), ())),
        preferred_element_type=jnp.float32,
    )
    m = jnp.max(s, axis=1, keepdims=True)
    p = jnp.exp(s - m)
    l = jnp.sum(p, axis=1, keepdims=True)
    o = jnp.dot(p.astype(jnp.bfloat16), v_ref[...],
                preferred_element_type=jnp.float32)
    o_ref[...] = (o / l).astype(jnp.bfloat16)


def _attn(q, kf, vf):
    return pl.pallas_call(
        _attn_body,
        grid=(HQ, SQ // QB),
        in_specs=[
            pl.BlockSpec((QB, DH), lambda h, qb: (qb, h)),
            pl.BlockSpec((N_DEV * SQ, DH), lambda h, qb: (0, h)),
            pl.BlockSpec((N_DEV * SQ, DH), lambda h, qb: (0, h)),
        ],
        out_specs=pl.BlockSpec((QB, DH), lambda h, qb: (qb, h)),
        out_shape=jax.ShapeDtypeStruct((SQ, D), jnp.bfloat16),
        compiler_params=pltpu.CompilerParams(
            vmem_limit_bytes=60 * 1024 * 1024),
    )(q, kf, vf)


def _proj_body(c_ref, w_ref, o_ref):
    o_ref[...] = jnp.dot(
        c_ref[...], w_ref[...].astype(jnp.bfloat16),
        preferred_element_type=jnp.float32)


def _proj(ctx, Wo):
    return pl.pallas_call(
        _proj_body,
        out_shape=jax.ShapeDtypeStruct((SQ, D), jnp.float32),
    )(ctx, Wo)


def kernel(x, Wq, Wk, Wv, Wo):
    x2 = x.reshape(SQ, D)
    q, k, v = _qkv(x2, Wq, Wk, Wv)
    kf, vf = _ring_gather(k, v)
    ctx = _attn(q, kf, vf)
    out = _proj(ctx, Wo)
    return out.reshape(1, SQ, D)


# device time: 438203 ns/iter; 2.3588x vs baseline; 1.4372x over previous
import math

import jax
import jax.numpy as jnp
from jax import lax
from jax.experimental import pallas as pl
from jax.experimental.pallas import tpu as pltpu

N_DEV = 4
SQ = 2048
D = 1024
HQ = 8
DH = 128
QB = 512
SCALE = 0.08838834764831843


def _rope_tables(off):
    ri = lax.broadcasted_iota(jnp.int32, (SQ, DH), 0).astype(jnp.float32)
    ci = lax.broadcasted_iota(jnp.int32, (SQ, DH), 1)
    f = (ci // 2).astype(jnp.float32)
    inv = jnp.exp(f * (-math.log(10000.0) / (DH // 2)))
    ang = (off.astype(jnp.float32) + ri) * inv
    return jnp.cos(ang), jnp.sin(ang)


def _rot_mat():
    i = lax.broadcasted_iota(jnp.int32, (DH, DH), 0)
    j = lax.broadcasted_iota(jnp.int32, (DH, DH), 1)
    plus = (j == i + 1) & (i % 2 == 0)
    minus = (j == i - 1) & (i % 2 == 1)
    return plus.astype(jnp.float32) - minus.astype(jnp.float32)


def _qkv_body(x_ref, wq_ref, wk_ref, wv_ref, q_ref, k_ref, v_ref):
    off = lax.axis_index("i") * SQ
    cos, sin = _rope_tables(off)
    rot = _rot_mat().astype(jnp.bfloat16)
    x = x_ref[...].astype(jnp.bfloat16)

    def proj_rope(w_ref):
        t = jnp.dot(x, w_ref[...].astype(jnp.bfloat16),
                    preferred_element_type=jnp.float32)
        tr = jnp.dot(t.astype(jnp.bfloat16), rot,
                     preferred_element_type=jnp.float32)
        return t * cos + tr * sin

    q_ref[0] = (proj_rope(wq_ref) * SCALE).astype(jnp.bfloat16)
    k_ref[0] = proj_rope(wk_ref).astype(jnp.bfloat16)
    v_ref[0] = jnp.dot(x, wv_ref[...].astype(jnp.bfloat16),
                       preferred_element_type=jnp.float32
                       ).astype(jnp.bfloat16)


def _qkv(x, Wq, Wk, Wv):
    return pl.pallas_call(
        _qkv_body,
        grid=(HQ,),
        in_specs=[
            pl.BlockSpec((SQ, D), lambda h: (0, 0)),
            pl.BlockSpec((D, DH), lambda h: (0, h)),
            pl.BlockSpec((D, DH), lambda h: (0, h)),
            pl.BlockSpec((D, DH), lambda h: (0, h)),
        ],
        out_specs=[pl.BlockSpec((1, SQ, DH), lambda h: (h, 0, 0))] * 3,
        out_shape=[jax.ShapeDtypeStruct((HQ, SQ, DH), jnp.bfloat16)] * 3,
    )(x, Wq, Wk, Wv)


def _fused_body(q_ref, k_ref, v_ref, o_ref,
                ck_ref, cv_ref, l_ref, sk, rk, sv, rv):
    my = lax.axis_index("i")
    left = lax.rem(my + N_DEV - 1, N_DEV)
    right = lax.rem(my + 1, N_DEV)

    barrier = pltpu.get_barrier_semaphore()
    for nbr in (left, right):
        pl.semaphore_signal(
            barrier, inc=1, device_id=(nbr,),
            device_id_type=pl.DeviceIdType.MESH,
        )
    pl.semaphore_wait(barrier, 2)

    for c in range(N_DEV):
        kc = k_ref if c == 0 else ck_ref.at[c - 1]
        vc = v_ref if c == 0 else cv_ref.at[c - 1]
        if c < N_DEV - 1:
            rdma_k = pltpu.make_async_remote_copy(
                src_ref=kc, dst_ref=ck_ref.at[c],
                send_sem=sk.at[c], recv_sem=rk.at[c],
                device_id=(right,), device_id_type=pl.DeviceIdType.MESH,
            )
            rdma_v = pltpu.make_async_remote_copy(
                src_ref=vc, dst_ref=cv_ref.at[c],
                send_sem=sv.at[c], recv_sem=rv.at[c],
                device_id=(right,), device_id_type=pl.DeviceIdType.MESH,
            )
            rdma_k.start()
            rdma_v.start()

        def head_body(h, _, kc=kc, vc=vc, first=(c == 0)):
            kh = kc[h]
            vh = vc[h]
            for qb in range(SQ // QB):
                qs = qb * QB
                qh = q_ref[h, pl.ds(qs, QB), :]
                s = lax.dot_general(
                    qh, kh, (((1,), (1,)), ((), ())),
                    preferred_element_type=jnp.float32,
                )
                p = jnp.exp(s)
                lsum = jnp.broadcast_to(
                    jnp.sum(p, axis=1, keepdims=True), (QB, DH))
                pv = jnp.dot(p.astype(jnp.bfloat16), vh,
                             preferred_element_type=jnp.float32)
                if first:
                    o_ref[h, pl.ds(qs, QB), :] = pv
                    l_ref[h, pl.ds(qs, QB), :] = lsum
                else:
                    o_ref[h, pl.ds(qs, QB), :] = (
                        o_ref[h, pl.ds(qs, QB), :] + pv)
                    l_ref[h, pl.ds(qs, QB), :] = (
                        l_ref[h, pl.ds(qs, QB), :] + lsum)
            return 0

        lax.fori_loop(0, HQ, head_body, 0)

        if c < N_DEV - 1:
            rdma_k.wait()
            rdma_v.wait()

    def norm_body(h, _):
        for qb in range(SQ // QB):
            qs = qb * QB
            o_ref[h, pl.ds(qs, QB), :] = (
                o_ref[h, pl.ds(qs, QB), :] / l_ref[h, pl.ds(qs, QB), :])
        return 0

    lax.fori_loop(0, HQ, norm_body, 0)


def _fused(q, k, v):
    return pl.pallas_call(
        _fused_body,
        in_specs=[pl.BlockSpec(memory_space=pltpu.MemorySpace.VMEM)] * 3,
        out_specs=pl.BlockSpec(memory_space=pltpu.MemorySpace.VMEM),
        out_shape=jax.ShapeDtypeStruct((HQ, SQ, DH), jnp.float32),
        scratch_shapes=[
            pltpu.VMEM((N_DEV - 1, HQ, SQ, DH), jnp.bfloat16),
            pltpu.VMEM((N_DEV - 1, HQ, SQ, DH), jnp.bfloat16),
            pltpu.VMEM((HQ, SQ, DH), jnp.float32),
            pltpu.SemaphoreType.DMA((N_DEV - 1,)),
            pltpu.SemaphoreType.DMA((N_DEV - 1,)),
            pltpu.SemaphoreType.DMA((N_DEV - 1,)),
            pltpu.SemaphoreType.DMA((N_DEV - 1,)),
        ],
        compiler_params=pltpu.CompilerParams(
            collective_id=0,
            vmem_limit_bytes=62 * 1024 * 1024,
        ),
    )(q, k, v)


def _proj_body(c_ref, w_ref, o_ref):
    h = pl.program_id(0)
    ctx = c_ref[0].astype(jnp.bfloat16)
    part = jnp.dot(ctx, w_ref[...].astype(jnp.bfloat16),
                   preferred_element_type=jnp.float32)

    @pl.when(h == 0)
    def _():
        o_ref[...] = part

    @pl.when(h > 0)
    def _():
        o_ref[...] += part


def _proj(ctx, Wo):
    return pl.pallas_call(
        _proj_body,
        grid=(HQ,),
        in_specs=[
            pl.BlockSpec((1, SQ, DH), lambda h: (h, 0, 0)),
            pl.BlockSpec((DH, D), lambda h: (h, 0)),
        ],
        out_specs=pl.BlockSpec((SQ, D), lambda h: (0, 0)),
        out_shape=jax.ShapeDtypeStruct((SQ, D), jnp.float32),
    )(ctx, Wo)


def kernel(x, Wq, Wk, Wv, Wo):
    x2 = x.reshape(SQ, D)
    q, k, v = _qkv(x2, Wq, Wk, Wv)
    ctx = _fused(q, k, v)
    out = _proj(ctx, Wo)
    return out.reshape(1, SQ, D)


# device time: 417956 ns/iter; 2.4730x vs baseline; 1.0484x over previous
import math

import jax
import jax.numpy as jnp
from jax import lax
from jax.experimental import pallas as pl
from jax.experimental.pallas import tpu as pltpu

N_DEV = 4
SQ = 2048
D = 1024
HQ = 8
DH = 128
QB = 512
SCALE = 0.08838834764831843


def _rope_tables(off):
    ri = lax.broadcasted_iota(jnp.int32, (SQ, DH), 0).astype(jnp.float32)
    ci = lax.broadcasted_iota(jnp.int32, (SQ, DH), 1)
    f = (ci // 2).astype(jnp.float32)
    inv = jnp.exp(f * (-math.log(10000.0) / (DH // 2)))
    ang = (off.astype(jnp.float32) + ri) * inv
    return jnp.cos(ang), jnp.sin(ang)


def _rot_mat():
    i = lax.broadcasted_iota(jnp.int32, (DH, DH), 0)
    j = lax.broadcasted_iota(jnp.int32, (DH, DH), 1)
    plus = (j == i + 1) & (i % 2 == 0)
    minus = (j == i - 1) & (i % 2 == 1)
    return plus.astype(jnp.float32) - minus.astype(jnp.float32)


def _qkv_body(x_ref, wq_ref, wk_ref, wv_ref, q_ref, k_ref, v_ref,
              cos_ref, sin_ref):
    @pl.when(pl.program_id(0) == 0)
    def _():
        off = lax.axis_index("i") * SQ
        c, s = _rope_tables(off)
        cos_ref[...] = c
        sin_ref[...] = s

    cos = cos_ref[...]
    sin = sin_ref[...]
    rot = _rot_mat().astype(jnp.bfloat16)
    x = x_ref[...].astype(jnp.bfloat16)

    def proj_rope(w_ref):
        t = jnp.dot(x, w_ref[...].astype(jnp.bfloat16),
                    preferred_element_type=jnp.float32)
        tr = jnp.dot(t.astype(jnp.bfloat16), rot,
                     preferred_element_type=jnp.float32)
        return t * cos + tr * sin

    q_ref[0] = (proj_rope(wq_ref) * SCALE).astype(jnp.bfloat16)
    k_ref[0] = proj_rope(wk_ref).astype(jnp.bfloat16)
    v_ref[0] = jnp.dot(x, wv_ref[...].astype(jnp.bfloat16),
                       preferred_element_type=jnp.float32
                       ).astype(jnp.bfloat16)


def _qkv(x, Wq, Wk, Wv):
    return pl.pallas_call(
        _qkv_body,
        grid=(HQ,),
        in_specs=[
            pl.BlockSpec((SQ, D), lambda h: (0, 0)),
            pl.BlockSpec((D, DH), lambda h: (0, h)),
            pl.BlockSpec((D, DH), lambda h: (0, h)),
            pl.BlockSpec((D, DH), lambda h: (0, h)),
        ],
        out_specs=[pl.BlockSpec((1, SQ, DH), lambda h: (h, 0, 0))] * 3,
        out_shape=[jax.ShapeDtypeStruct((HQ, SQ, DH), jnp.bfloat16)] * 3,
        scratch_shapes=[
            pltpu.VMEM((SQ, DH), jnp.float32),
            pltpu.VMEM((SQ, DH), jnp.float32),
        ],
    )(x, Wq, Wk, Wv)


def _fused_body(q_ref, k_ref, v_ref, o_ref,
                ck_ref, cv_ref, l_ref, sk, rk, sv, rv):
    my = lax.axis_index("i")
    left = lax.rem(my + N_DEV - 1, N_DEV)
    right = lax.rem(my + 1, N_DEV)

    barrier = pltpu.get_barrier_semaphore()
    for nbr in (left, right):
        pl.semaphore_signal(
            barrier, inc=1, device_id=(nbr,),
            device_id_type=pl.DeviceIdType.MESH,
        )
    pl.semaphore_wait(barrier, 2)

    for c in range(N_DEV):
        kc = k_ref if c == 0 else ck_ref.at[c - 1]
        vc = v_ref if c == 0 else cv_ref.at[c - 1]
        if c < N_DEV - 1:
            rdma_k = pltpu.make_async_remote_copy(
                src_ref=kc, dst_ref=ck_ref.at[c],
                send_sem=sk.at[c], recv_sem=rk.at[c],
                device_id=(right,), device_id_type=pl.DeviceIdType.MESH,
            )
            rdma_v = pltpu.make_async_remote_copy(
                src_ref=vc, dst_ref=cv_ref.at[c],
                send_sem=sv.at[c], recv_sem=rv.at[c],
                device_id=(right,), device_id_type=pl.DeviceIdType.MESH,
            )
            rdma_k.start()
            rdma_v.start()

        def head_body(h, _, kc=kc, vc=vc, first=(c == 0)):
            kh = kc[h]
            vh = vc[h]
            for qb in range(SQ // QB):
                qs = qb * QB
                qh = q_ref[h, pl.ds(qs, QB), :]
                s = lax.dot_general(
                    qh, kh, (((1,), (1,)), ((), ())),
                    preferred_element_type=jnp.float32,
                )
                p = jnp.exp(s.astype(jnp.bfloat16))
                lsum = jnp.broadcast_to(
                    jnp.sum(p, axis=1, keepdims=True,
                            dtype=jnp.float32), (QB, DH))
                pv = jnp.dot(p, vh, preferred_element_type=jnp.float32)
                if first:
                    o_ref[h, pl.ds(qs, QB), :] = pv
                    l_ref[h, pl.ds(qs, QB), :] = lsum
                else:
                    o_ref[h, pl.ds(qs, QB), :] = (
                        o_ref[h, pl.ds(qs, QB), :] + pv)
                    l_ref[h, pl.ds(qs, QB), :] = (
                        l_ref[h, pl.ds(qs, QB), :] + lsum)
            return 0

        lax.fori_loop(0, HQ, head_body, 0)

        if c < N_DEV - 1:
            rdma_k.wait()
            rdma_v.wait()

    def norm_body(h, _):
        for qb in range(SQ // QB):
            qs = qb * QB
            o_ref[h, pl.ds(qs, QB), :] = (
                o_ref[h, pl.ds(qs, QB), :] / l_ref[h, pl.ds(qs, QB), :])
        return 0

    lax.fori_loop(0, HQ, norm_body, 0)


def _fused(q, k, v):
    return pl.pallas_call(
        _fused_body,
        in_specs=[pl.BlockSpec(memory_space=pltpu.MemorySpace.VMEM)] * 3,
        out_specs=pl.BlockSpec(memory_space=pltpu.MemorySpace.VMEM),
        out_shape=jax.ShapeDtypeStruct((HQ, SQ, DH), jnp.float32),
        scratch_shapes=[
            pltpu.VMEM((N_DEV - 1, HQ, SQ, DH), jnp.bfloat16),
            pltpu.VMEM((N_DEV - 1, HQ, SQ, DH), jnp.bfloat16),
            pltpu.VMEM((HQ, SQ, DH), jnp.float32),
            pltpu.SemaphoreType.DMA((N_DEV - 1,)),
            pltpu.SemaphoreType.DMA((N_DEV - 1,)),
            pltpu.SemaphoreType.DMA((N_DEV - 1,)),
            pltpu.SemaphoreType.DMA((N_DEV - 1,)),
        ],
        compiler_params=pltpu.CompilerParams(
            collective_id=0,
            vmem_limit_bytes=62 * 1024 * 1024,
        ),
    )(q, k, v)


def _proj_body(c_ref, w_ref, o_ref):
    h = pl.program_id(0)
    ctx = c_ref[0].astype(jnp.bfloat16)
    part = jnp.dot(ctx, w_ref[...].astype(jnp.bfloat16),
                   preferred_element_type=jnp.float32)

    @pl.when(h == 0)
    def _():
        o_ref[...] = part

    @pl.when(h > 0)
    def _():
        o_ref[...] += part


def _proj(ctx, Wo):
    return pl.pallas_call(
        _proj_body,
        grid=(HQ,),
        in_specs=[
            pl.BlockSpec((1, SQ, DH), lambda h: (h, 0, 0)),
            pl.BlockSpec((DH, D), lambda h: (h, 0)),
        ],
        out_specs=pl.BlockSpec((SQ, D), lambda h: (0, 0)),
        out_shape=jax.ShapeDtypeStruct((SQ, D), jnp.float32),
    )(ctx, Wo)


def kernel(x, Wq, Wk, Wv, Wo):
    x2 = x.reshape(SQ, D)
    q, k, v = _qkv(x2, Wq, Wk, Wv)
    ctx = _fused(q, k, v)
    out = _proj(ctx, Wo)
    return out.reshape(1, SQ, D)


# device time: 311963 ns/iter; 3.3133x vs baseline; 1.3398x over previous
import math

import jax
import jax.numpy as jnp
from jax import lax
from jax.experimental import pallas as pl
from jax.experimental.pallas import tpu as pltpu

N_DEV = 4
SQ = 2048
D = 1024
HQ = 8
DH = 128
QB = 512
SCALE = 0.08838834764831843


def _rope_tables(off):
    ri = lax.broadcasted_iota(jnp.int32, (SQ, DH), 0).astype(jnp.float32)
    ci = lax.broadcasted_iota(jnp.int32, (SQ, DH), 1)
    f = (ci // 2).astype(jnp.float32)
    inv = jnp.exp(f * (-math.log(10000.0) / (DH // 2)))
    ang = (off.astype(jnp.float32) + ri) * inv
    return jnp.cos(ang), jnp.sin(ang)


def _rot_mat():
    i = lax.broadcasted_iota(jnp.int32, (DH, DH), 0)
    j = lax.broadcasted_iota(jnp.int32, (DH, DH), 1)
    plus = (j == i + 1) & (i % 2 == 0)
    minus = (j == i - 1) & (i % 2 == 1)
    return plus.astype(jnp.float32) - minus.astype(jnp.float32)


def _qkv_body(x_ref, wq_ref, wk_ref, wv_ref, q_ref, k_ref, v_ref,
              cos_ref, sin_ref):
    @pl.when(pl.program_id(0) == 0)
    def _():
        off = lax.axis_index("i") * SQ
        c, s = _rope_tables(off)
        cos_ref[...] = c
        sin_ref[...] = s

    cos = cos_ref[...]
    sin = sin_ref[...]
    rot = _rot_mat().astype(jnp.bfloat16)
    x = x_ref[...].astype(jnp.bfloat16)

    def proj_rope(w_ref):
        t = jnp.dot(x, w_ref[...].astype(jnp.bfloat16),
                    preferred_element_type=jnp.float32)
        tr = jnp.dot(t.astype(jnp.bfloat16), rot,
                     preferred_element_type=jnp.float32)
        return t * cos + tr * sin

    q_ref[0] = (proj_rope(wq_ref) * SCALE).astype(jnp.bfloat16)
    k_ref[0] = proj_rope(wk_ref).astype(jnp.bfloat16)
    v_ref[0] = jnp.dot(x, wv_ref[...].astype(jnp.bfloat16),
                       preferred_element_type=jnp.float32
                       ).astype(jnp.bfloat16)


def _qkv(x, Wq, Wk, Wv):
    return pl.pallas_call(
        _qkv_body,
        grid=(HQ,),
        in_specs=[
            pl.BlockSpec((SQ, D), lambda h: (0, 0)),
            pl.BlockSpec((D, DH), lambda h: (0, h)),
            pl.BlockSpec((D, DH), lambda h: (0, h)),
            pl.BlockSpec((D, DH), lambda h: (0, h)),
        ],
        out_specs=[pl.BlockSpec((1, SQ, DH), lambda h: (h, 0, 0))] * 3,
        out_shape=[jax.ShapeDtypeStruct((HQ, SQ, DH), jnp.bfloat16)] * 3,
        scratch_shapes=[
            pltpu.VMEM((SQ, DH), jnp.float32),
            pltpu.VMEM((SQ, DH), jnp.float32),
        ],
    )(x, Wq, Wk, Wv)


def _fused_body(q_ref, k_ref, v_ref, o_ref,
                ckL, cvL, ckR, cvR, cko, cvo, l_ref, ssem, rsem):
    my = lax.axis_index("i")
    left = lax.rem(my + N_DEV - 1, N_DEV)
    right = lax.rem(my + 1, N_DEV)

    barrier = pltpu.get_barrier_semaphore()
    for nbr in (left, right):
        pl.semaphore_signal(
            barrier, inc=1, device_id=(nbr,),
            device_id_type=pl.DeviceIdType.MESH,
        )
    pl.semaphore_wait(barrier, 2)

    H2 = HQ // 2

    def rc(src, dst, i, dev):
        return pltpu.make_async_remote_copy(
            src_ref=src, dst_ref=dst,
            send_sem=ssem.at[i], recv_sem=rsem.at[i],
            device_id=(dev,), device_id_type=pl.DeviceIdType.MESH,
        )

    def attn_chunk(kc, vc, first=False):
        def head_body(h, _):
            kh = kc[h]
            vh = vc[h]
            for qb in range(SQ // QB):
                qs = qb * QB
                qh = q_ref[h, pl.ds(qs, QB), :]
                s = lax.dot_general(
                    qh, kh, (((1,), (1,)), ((), ())),
                    preferred_element_type=jnp.float32,
                )
                p = jnp.exp(s.astype(jnp.bfloat16))
                lsum = jnp.broadcast_to(
                    jnp.sum(p, axis=1, keepdims=True,
                            dtype=jnp.float32), (QB, DH))
                pv = jnp.dot(p, vh, preferred_element_type=jnp.float32)
                if first:
                    o_ref[h, pl.ds(qs, QB), :] = pv
                    l_ref[h, pl.ds(qs, QB), :] = lsum
                else:
                    o_ref[h, pl.ds(qs, QB), :] = (
                        o_ref[h, pl.ds(qs, QB), :] + pv)
                    l_ref[h, pl.ds(qs, QB), :] = (
                        l_ref[h, pl.ds(qs, QB), :] + lsum)
            return 0

        lax.fori_loop(0, HQ, head_body, 0)

    d1 = rc(k_ref, ckL, 0, right)
    d2 = rc(v_ref, cvL, 1, right)
    d3 = rc(k_ref, ckR, 2, left)
    d4 = rc(v_ref, cvR, 3, left)
    d1.start()
    d2.start()
    d3.start()
    d4.start()

    attn_chunk(k_ref, v_ref, first=True)

    d1.wait()
    d2.wait()
    d5 = rc(ckL.at[pl.ds(0, H2)], cko.at[pl.ds(0, H2)], 4, right)
    d6 = rc(cvL.at[pl.ds(0, H2)], cvo.at[pl.ds(0, H2)], 5, right)
    d5.start()
    d6.start()
    d3.wait()
    d4.wait()
    d7 = rc(ckR.at[pl.ds(H2, H2)], cko.at[pl.ds(H2, H2)], 6, left)
    d8 = rc(cvR.at[pl.ds(H2, H2)], cvo.at[pl.ds(H2, H2)], 7, left)
    d7.start()
    d8.start()

    attn_chunk(ckL, cvL)
    attn_chunk(ckR, cvR)

    d5.wait()
    d6.wait()
    d7.wait()
    d8.wait()

    attn_chunk(cko, cvo)

    def norm_body(h, _):
        for qb in range(SQ // QB):
            qs = qb * QB
            o_ref[h, pl.ds(qs, QB), :] = (
                o_ref[h, pl.ds(qs, QB), :] / l_ref[h, pl.ds(qs, QB), :])
        return 0

    lax.fori_loop(0, HQ, norm_body, 0)


def _fused(q, k, v):
    return pl.pallas_call(
        _fused_body,
        in_specs=[pl.BlockSpec(memory_space=pltpu.MemorySpace.VMEM)] * 3,
        out_specs=pl.BlockSpec(memory_space=pltpu.MemorySpace.VMEM),
        out_shape=jax.ShapeDtypeStruct((HQ, SQ, DH), jnp.float32),
        scratch_shapes=[
            pltpu.VMEM((HQ, SQ, DH), jnp.bfloat16),
            pltpu.VMEM((HQ, SQ, DH), jnp.bfloat16),
            pltpu.VMEM((HQ, SQ, DH), jnp.bfloat16),
            pltpu.VMEM((HQ, SQ, DH), jnp.bfloat16),
            pltpu.VMEM((HQ, SQ, DH), jnp.bfloat16),
            pltpu.VMEM((HQ, SQ, DH), jnp.bfloat16),
            pltpu.VMEM((HQ, SQ, DH), jnp.float32),
            pltpu.SemaphoreType.DMA((8,)),
            pltpu.SemaphoreType.DMA((8,)),
        ],
        compiler_params=pltpu.CompilerParams(
            collective_id=0,
            vmem_limit_bytes=62 * 1024 * 1024,
        ),
    )(q, k, v)


def _proj_body(c_ref, w_ref, o_ref):
    h = pl.program_id(0)
    ctx = c_ref[0].astype(jnp.bfloat16)
    part = jnp.dot(ctx, w_ref[...].astype(jnp.bfloat16),
                   preferred_element_type=jnp.float32)

    @pl.when(h == 0)
    def _():
        o_ref[...] = part

    @pl.when(h > 0)
    def _():
        o_ref[...] += part


def _proj(ctx, Wo):
    return pl.pallas_call(
        _proj_body,
        grid=(HQ,),
        in_specs=[
            pl.BlockSpec((1, SQ, DH), lambda h: (h, 0, 0)),
            pl.BlockSpec((DH, D), lambda h: (h, 0)),
        ],
        out_specs=pl.BlockSpec((SQ, D), lambda h: (0, 0)),
        out_shape=jax.ShapeDtypeStruct((SQ, D), jnp.float32),
    )(ctx, Wo)


def kernel(x, Wq, Wk, Wv, Wo):
    x2 = x.reshape(SQ, D)
    q, k, v = _qkv(x2, Wq, Wk, Wv)
    ctx = _fused(q, k, v)
    out = _proj(ctx, Wo)
    return out.reshape(1, SQ, D)


# device time: 276130 ns/iter; 3.7433x vs baseline; 1.1298x over previous
import math

import jax
import jax.numpy as jnp
from jax import lax
from jax.experimental import pallas as pl
from jax.experimental.pallas import tpu as pltpu

N_DEV = 4
SQ = 2048
D = 1024
HQ = 8
DH = 128
QB = 512
SCALE = 0.08838834764831843


def _rope_tables(off):
    ri = lax.broadcasted_iota(jnp.int32, (SQ, DH), 0).astype(jnp.float32)
    ci = lax.broadcasted_iota(jnp.int32, (SQ, DH), 1)
    f = (ci // 2).astype(jnp.float32)
    inv = jnp.exp(f * (-math.log(10000.0) / (DH // 2)))
    ang = (off.astype(jnp.float32) + ri) * inv
    return jnp.cos(ang), jnp.sin(ang)


def _rot_mat():
    i = lax.broadcasted_iota(jnp.int32, (DH, DH), 0)
    j = lax.broadcasted_iota(jnp.int32, (DH, DH), 1)
    plus = (j == i + 1) & (i % 2 == 0)
    minus = (j == i - 1) & (i % 2 == 1)
    return plus.astype(jnp.float32) - minus.astype(jnp.float32)


_HG = 4
_WG = _HG * DH


def _qkv_body(x_ref, wq_ref, wk_ref, wv_ref, q_ref, k_ref, v_ref,
              cos_ref, sin_ref):
    @pl.when(pl.program_id(0) == 0)
    def _():
        off = lax.axis_index("i") * SQ
        c, s = _rope_tables(off)
        cos_ref[...] = c
        sin_ref[...] = s

    cos = cos_ref[...]
    sin = sin_ref[...]
    rot = _rot_mat().astype(jnp.bfloat16)
    x = x_ref[...].astype(jnp.bfloat16)

    tq = jnp.dot(x, wq_ref[...].astype(jnp.bfloat16),
                 preferred_element_type=jnp.float32)
    tk = jnp.dot(x, wk_ref[...].astype(jnp.bfloat16),
                 preferred_element_type=jnp.float32)
    tv = jnp.dot(x, wv_ref[...].astype(jnp.bfloat16),
                 preferred_element_type=jnp.float32)
    for hh in range(_HG):
        t = tq[:, hh * DH:(hh + 1) * DH]
        tr = jnp.dot(t.astype(jnp.bfloat16), rot,
                     preferred_element_type=jnp.float32)
        q_ref[hh] = ((t * cos + tr * sin) * SCALE).astype(jnp.bfloat16)
        t = tk[:, hh * DH:(hh + 1) * DH]
        tr = jnp.dot(t.astype(jnp.bfloat16), rot,
                     preferred_element_type=jnp.float32)
        k_ref[hh] = (t * cos + tr * sin).astype(jnp.bfloat16)
        v_ref[hh] = tv[:, hh * DH:(hh + 1) * DH].astype(jnp.bfloat16)


def _qkv(x, Wq, Wk, Wv):
    return pl.pallas_call(
        _qkv_body,
        grid=(HQ // _HG,),
        in_specs=[
            pl.BlockSpec((SQ, D), lambda g: (0, 0)),
            pl.BlockSpec((D, _WG), lambda g: (0, g)),
            pl.BlockSpec((D, _WG), lambda g: (0, g)),
            pl.BlockSpec((D, _WG), lambda g: (0, g)),
        ],
        out_specs=[pl.BlockSpec((_HG, SQ, DH), lambda g: (g, 0, 0))] * 3,
        out_shape=[jax.ShapeDtypeStruct((HQ, SQ, DH), jnp.bfloat16)] * 3,
        scratch_shapes=[
            pltpu.VMEM((SQ, DH), jnp.float32),
            pltpu.VMEM((SQ, DH), jnp.float32),
        ],
        compiler_params=pltpu.CompilerParams(
            vmem_limit_bytes=60 * 1024 * 1024),
    )(x, Wq, Wk, Wv)


def _fused_body(q_ref, k_ref, v_ref, o_ref,
                ckL, cvL, ckR, cvR, cko, cvo, l_ref, ssem, rsem):
    my = lax.axis_index("i")
    left = lax.rem(my + N_DEV - 1, N_DEV)
    right = lax.rem(my + 1, N_DEV)

    barrier = pltpu.get_barrier_semaphore()
    for nbr in (left, right):
        pl.semaphore_signal(
            barrier, inc=1, device_id=(nbr,),
            device_id_type=pl.DeviceIdType.MESH,
        )
    pl.semaphore_wait(barrier, 2)

    H2 = HQ // 2

    def rc(src, dst, i, dev):
        return pltpu.make_async_remote_copy(
            src_ref=src, dst_ref=dst,
            send_sem=ssem.at[i], recv_sem=rsem.at[i],
            device_id=(dev,), device_id_type=pl.DeviceIdType.MESH,
        )

    def attn_chunk(kc, vc, first=False):
        def head_body(h, _):
            kh = kc[h]
            vh = vc[h]
            for qb in range(SQ // QB):
                qs = qb * QB
                qh = q_ref[h, pl.ds(qs, QB), :]
                s = lax.dot_general(
                    qh, kh, (((1,), (1,)), ((), ())),
                    preferred_element_type=jnp.float32,
                )
                p = jnp.exp(s.astype(jnp.bfloat16))
                lsum = jnp.broadcast_to(
                    jnp.sum(p, axis=1, keepdims=True,
                            dtype=jnp.float32), (QB, DH))
                pv = jnp.dot(p, vh, preferred_element_type=jnp.float32)
                if first:
                    o_ref[h, pl.ds(qs, QB), :] = pv
                    l_ref[h, pl.ds(qs, QB), :] = lsum
                else:
                    o_ref[h, pl.ds(qs, QB), :] = (
                        o_ref[h, pl.ds(qs, QB), :] + pv)
                    l_ref[h, pl.ds(qs, QB), :] = (
                        l_ref[h, pl.ds(qs, QB), :] + lsum)
            return 0

        lax.fori_loop(0, HQ, head_body, 0)

    d1 = rc(k_ref, ckL, 0, right)
    d2 = rc(v_ref, cvL, 1, right)
    d3 = rc(k_ref, ckR, 2, left)
    d4 = rc(v_ref, cvR, 3, left)
    d1.start()
    d2.start()
    d3.start()
    d4.start()

    attn_chunk(k_ref, v_ref, first=True)

    d1.wait()
    d2.wait()
    d5 = rc(ckL.at[pl.ds(0, H2)], cko.at[pl.ds(0, H2)], 4, right)
    d6 = rc(cvL.at[pl.ds(0, H2)], cvo.at[pl.ds(0, H2)], 5, right)
    d5.start()
    d6.start()
    d3.wait()
    d4.wait()
    d7 = rc(ckR.at[pl.ds(H2, H2)], cko.at[pl.ds(H2, H2)], 6, left)
    d8 = rc(cvR.at[pl.ds(H2, H2)], cvo.at[pl.ds(H2, H2)], 7, left)
    d7.start()
    d8.start()

    attn_chunk(ckL, cvL)
    attn_chunk(ckR, cvR)

    d5.wait()
    d6.wait()
    d7.wait()
    d8.wait()

    attn_chunk(cko, cvo)

    def norm_body(h, _):
        for qb in range(SQ // QB):
            qs = qb * QB
            o_ref[h, pl.ds(qs, QB), :] = (
                o_ref[h, pl.ds(qs, QB), :] / l_ref[h, pl.ds(qs, QB), :])
        return 0

    lax.fori_loop(0, HQ, norm_body, 0)


def _fused(q, k, v):
    return pl.pallas_call(
        _fused_body,
        in_specs=[pl.BlockSpec(memory_space=pltpu.MemorySpace.VMEM)] * 3,
        out_specs=pl.BlockSpec(memory_space=pltpu.MemorySpace.VMEM),
        out_shape=jax.ShapeDtypeStruct((HQ, SQ, DH), jnp.float32),
        scratch_shapes=[
            pltpu.VMEM((HQ, SQ, DH), jnp.bfloat16),
            pltpu.VMEM((HQ, SQ, DH), jnp.bfloat16),
            pltpu.VMEM((HQ, SQ, DH), jnp.bfloat16),
            pltpu.VMEM((HQ, SQ, DH), jnp.bfloat16),
            pltpu.VMEM((HQ, SQ, DH), jnp.bfloat16),
            pltpu.VMEM((HQ, SQ, DH), jnp.bfloat16),
            pltpu.VMEM((HQ, SQ, DH), jnp.float32),
            pltpu.SemaphoreType.DMA((8,)),
            pltpu.SemaphoreType.DMA((8,)),
        ],
        compiler_params=pltpu.CompilerParams(
            collective_id=0,
            vmem_limit_bytes=62 * 1024 * 1024,
        ),
    )(q, k, v)


def _proj_body(c_ref, w_ref, o_ref):
    h = pl.program_id(0)
    ctx = c_ref[0].astype(jnp.bfloat16)
    part = jnp.dot(ctx, w_ref[...].astype(jnp.bfloat16),
                   preferred_element_type=jnp.float32)

    @pl.when(h == 0)
    def _():
        o_ref[...] = part

    @pl.when(h > 0)
    def _():
        o_ref[...] += part


def _proj(ctx, Wo):
    return pl.pallas_call(
        _proj_body,
        grid=(HQ,),
        in_specs=[
            pl.BlockSpec((1, SQ, DH), lambda h: (h, 0, 0)),
            pl.BlockSpec((DH, D), lambda h: (h, 0)),
        ],
        out_specs=pl.BlockSpec((SQ, D), lambda h: (0, 0)),
        out_shape=jax.ShapeDtypeStruct((SQ, D), jnp.float32),
    )(ctx, Wo)


def kernel(x, Wq, Wk, Wv, Wo):
    x2 = x.reshape(SQ, D)
    q, k, v = _qkv(x2, Wq, Wk, Wv)
    ctx = _fused(q, k, v)
    out = _proj(ctx, Wo)
    return out.reshape(1, SQ, D)


# device time: 273673 ns/iter; 3.7769x vs baseline; 1.0090x over previous
import math

import jax
import jax.numpy as jnp
from jax import lax
from jax.experimental import pallas as pl
from jax.experimental.pallas import tpu as pltpu

N_DEV = 4
SQ = 2048
D = 1024
HQ = 8
DH = 128
QB = 512
SCALE = 0.08838834764831843


def _rope_tables(off):
    ri = lax.broadcasted_iota(jnp.int32, (SQ, DH), 0).astype(jnp.float32)
    ci = lax.broadcasted_iota(jnp.int32, (SQ, DH), 1)
    f = (ci // 2).astype(jnp.float32)
    inv = jnp.exp(f * (-math.log(10000.0) / (DH // 2)))
    ang = (off.astype(jnp.float32) + ri) * inv
    return jnp.cos(ang), jnp.sin(ang)


def _rot_mat():
    i = lax.broadcasted_iota(jnp.int32, (DH, DH), 0)
    j = lax.broadcasted_iota(jnp.int32, (DH, DH), 1)
    plus = (j == i + 1) & (i % 2 == 0)
    minus = (j == i - 1) & (i % 2 == 1)
    return plus.astype(jnp.float32) - minus.astype(jnp.float32)


_HG = 4
_WG = _HG * DH


def _qkv_body(x_ref, wq_ref, wk_ref, wv_ref, q_ref, k_ref, v_ref,
              cos_ref, sin_ref):
    @pl.when(pl.program_id(0) == 0)
    def _():
        off = lax.axis_index("i") * SQ
        c, s = _rope_tables(off)
        cos_ref[...] = c
        sin_ref[...] = s

    cos = cos_ref[...]
    sin = sin_ref[...]
    rot = _rot_mat().astype(jnp.bfloat16)
    x = x_ref[...].astype(jnp.bfloat16)

    tq = jnp.dot(x, wq_ref[...].astype(jnp.bfloat16),
                 preferred_element_type=jnp.float32)
    tk = jnp.dot(x, wk_ref[...].astype(jnp.bfloat16),
                 preferred_element_type=jnp.float32)
    tv = jnp.dot(x, wv_ref[...].astype(jnp.bfloat16),
                 preferred_element_type=jnp.float32)
    for hh in range(_HG):
        t = tq[:, hh * DH:(hh + 1) * DH]
        tr = jnp.dot(t.astype(jnp.bfloat16), rot,
                     preferred_element_type=jnp.float32)
        q_ref[hh] = ((t * cos + tr * sin) * SCALE).astype(jnp.bfloat16)
        t = tk[:, hh * DH:(hh + 1) * DH]
        tr = jnp.dot(t.astype(jnp.bfloat16), rot,
                     preferred_element_type=jnp.float32)
        k_ref[hh] = (t * cos + tr * sin).astype(jnp.bfloat16)
        v_ref[hh] = tv[:, hh * DH:(hh + 1) * DH].astype(jnp.bfloat16)


def _qkv(x, Wq, Wk, Wv):
    return pl.pallas_call(
        _qkv_body,
        grid=(HQ // _HG,),
        in_specs=[
            pl.BlockSpec((SQ, D), lambda g: (0, 0)),
            pl.BlockSpec((D, _WG), lambda g: (0, g)),
            pl.BlockSpec((D, _WG), lambda g: (0, g)),
            pl.BlockSpec((D, _WG), lambda g: (0, g)),
        ],
        out_specs=[pl.BlockSpec((_HG, SQ, DH), lambda g: (g, 0, 0))] * 3,
        out_shape=[jax.ShapeDtypeStruct((HQ, SQ, DH), jnp.bfloat16)] * 3,
        scratch_shapes=[
            pltpu.VMEM((SQ, DH), jnp.float32),
            pltpu.VMEM((SQ, DH), jnp.float32),
        ],
        compiler_params=pltpu.CompilerParams(
            vmem_limit_bytes=60 * 1024 * 1024),
    )(x, Wq, Wk, Wv)


def _fused_body(q_ref, k_ref, v_ref, o_ref,
                ckL, cvL, ckR, cvR, cko, cvo, l_ref, ssem, rsem):
    my = lax.axis_index("i")
    left = lax.rem(my + N_DEV - 1, N_DEV)
    right = lax.rem(my + 1, N_DEV)

    barrier = pltpu.get_barrier_semaphore()
    for nbr in (left, right):
        pl.semaphore_signal(
            barrier, inc=1, device_id=(nbr,),
            device_id_type=pl.DeviceIdType.MESH,
        )
    pl.semaphore_wait(barrier, 2)

    H2 = HQ // 2

    def rc(src, dst, i, dev):
        return pltpu.make_async_remote_copy(
            src_ref=src, dst_ref=dst,
            send_sem=ssem.at[i], recv_sem=rsem.at[i],
            device_id=(dev,), device_id_type=pl.DeviceIdType.MESH,
        )

    def attn_chunk(kc, vc, first=False):
        def head_body(h, _):
            kh = kc[h]
            ve = jnp.concatenate(
                [vc[h], jnp.ones((SQ, DH), jnp.bfloat16)], axis=1)
            for qb in range(SQ // QB):
                qs = qb * QB
                qh = q_ref[h, pl.ds(qs, QB), :]
                s = lax.dot_general(
                    qh, kh, (((1,), (1,)), ((), ())),
                    preferred_element_type=jnp.float32,
                )
                p = jnp.exp(s.astype(jnp.bfloat16))
                pv_l = jnp.dot(p, ve, preferred_element_type=jnp.float32)
                pv = pv_l[:, :DH]
                lsum = pv_l[:, DH:]
                if first:
                    o_ref[h, pl.ds(qs, QB), :] = pv
                    l_ref[h, pl.ds(qs, QB), :] = lsum
                else:
                    o_ref[h, pl.ds(qs, QB), :] = (
                        o_ref[h, pl.ds(qs, QB), :] + pv)
                    l_ref[h, pl.ds(qs, QB), :] = (
                        l_ref[h, pl.ds(qs, QB), :] + lsum)
            return 0

        lax.fori_loop(0, HQ, head_body, 0)

    d1 = rc(k_ref, ckL, 0, right)
    d2 = rc(v_ref, cvL, 1, right)
    d3 = rc(k_ref, ckR, 2, left)
    d4 = rc(v_ref, cvR, 3, left)
    d1.start()
    d2.start()
    d3.start()
    d4.start()

    attn_chunk(k_ref, v_ref, first=True)

    d1.wait()
    d2.wait()
    d5 = rc(ckL.at[pl.ds(0, H2)], cko.at[pl.ds(0, H2)], 4, right)
    d6 = rc(cvL.at[pl.ds(0, H2)], cvo.at[pl.ds(0, H2)], 5, right)
    d5.start()
    d6.start()
    d3.wait()
    d4.wait()
    d7 = rc(ckR.at[pl.ds(H2, H2)], cko.at[pl.ds(H2, H2)], 6, left)
    d8 = rc(cvR.at[pl.ds(H2, H2)], cvo.at[pl.ds(H2, H2)], 7, left)
    d7.start()
    d8.start()

    attn_chunk(ckL, cvL)
    attn_chunk(ckR, cvR)

    d5.wait()
    d6.wait()
    d7.wait()
    d8.wait()

    attn_chunk(cko, cvo)

    def norm_body(h, _):
        for qb in range(SQ // QB):
            qs = qb * QB
            o_ref[h, pl.ds(qs, QB), :] = (
                o_ref[h, pl.ds(qs, QB), :] / l_ref[h, pl.ds(qs, QB), :])
        return 0

    lax.fori_loop(0, HQ, norm_body, 0)


def _fused(q, k, v):
    return pl.pallas_call(
        _fused_body,
        in_specs=[pl.BlockSpec(memory_space=pltpu.MemorySpace.VMEM)] * 3,
        out_specs=pl.BlockSpec(memory_space=pltpu.MemorySpace.VMEM),
        out_shape=jax.ShapeDtypeStruct((HQ, SQ, DH), jnp.float32),
        scratch_shapes=[
            pltpu.VMEM((HQ, SQ, DH), jnp.bfloat16),
            pltpu.VMEM((HQ, SQ, DH), jnp.bfloat16),
            pltpu.VMEM((HQ, SQ, DH), jnp.bfloat16),
            pltpu.VMEM((HQ, SQ, DH), jnp.bfloat16),
            pltpu.VMEM((HQ, SQ, DH), jnp.bfloat16),
            pltpu.VMEM((HQ, SQ, DH), jnp.bfloat16),
            pltpu.VMEM((HQ, SQ, DH), jnp.float32),
            pltpu.SemaphoreType.DMA((8,)),
            pltpu.SemaphoreType.DMA((8,)),
        ],
        compiler_params=pltpu.CompilerParams(
            collective_id=0,
            vmem_limit_bytes=62 * 1024 * 1024,
        ),
    )(q, k, v)


def _proj_body(c_ref, w_ref, o_ref):
    h = pl.program_id(0)
    ctx = c_ref[0].astype(jnp.bfloat16)
    part = jnp.dot(ctx, w_ref[...].astype(jnp.bfloat16),
                   preferred_element_type=jnp.float32)

    @pl.when(h == 0)
    def _():
        o_ref[...] = part

    @pl.when(h > 0)
    def _():
        o_ref[...] += part


def _proj(ctx, Wo):
    return pl.pallas_call(
        _proj_body,
        grid=(HQ,),
        in_specs=[
            pl.BlockSpec((1, SQ, DH), lambda h: (h, 0, 0)),
            pl.BlockSpec((DH, D), lambda h: (h, 0)),
        ],
        out_specs=pl.BlockSpec((SQ, D), lambda h: (0, 0)),
        out_shape=jax.ShapeDtypeStruct((SQ, D), jnp.float32),
    )(ctx, Wo)


def kernel(x, Wq, Wk, Wv, Wo):
    x2 = x.reshape(SQ, D)
    q, k, v = _qkv(x2, Wq, Wk, Wv)
    ctx = _fused(q, k, v)
    out = _proj(ctx, Wo)
    return out.reshape(1, SQ, D)
